# baseline (speedup 1.0000x reference)
import functools

import jax
import jax.numpy as jnp
from jax import lax
from jax.experimental import pallas as pl
from jax.experimental.pallas import tpu as pltpu
from jax.experimental.pallas import tpu_sc as plsc

BATCH = 4096
SEQ = 200
DIM = 64
B_TOTAL = BATCH * SEQ
NUM_CORES = 2
NUM_SUBCORES = 16
NW = NUM_CORES * NUM_SUBCORES
B_PER_W = B_TOTAL // NW
CHUNK = 128
N_CHUNKS = B_PER_W // CHUNK

_mesh = plsc.VectorSubcoreMesh(core_axis_name="c", subcore_axis_name="s")


@functools.partial(
    pl.kernel,
    mesh=_mesh,
    out_type=jax.ShapeDtypeStruct((B_TOTAL, DIM), jnp.float32),
    scratch_types=[
        pltpu.VMEM((N_CHUNKS, CHUNK), jnp.int32),
        pltpu.VMEM((CHUNK, DIM), jnp.float32),
        pltpu.SemaphoreType.DMA,
    ],
    compiler_params=pltpu.CompilerParams(use_tc_tiling_on_sc=False),
)
def _gather_kernel(idx_hbm, table_hbm, out_hbm, idx_v, rows_v, sem):
    wid = lax.axis_index("s") * NUM_CORES + lax.axis_index("c")
    pltpu.sync_copy(idx_hbm.at[wid], idx_v)
    base = wid * B_PER_W

    def body(j, carry):
        pltpu.async_copy(table_hbm.at[idx_v.at[j]], rows_v, sem).wait()
        pltpu.sync_copy(rows_v, out_hbm.at[pl.ds(base + j * CHUNK, CHUNK)])
        return carry

    lax.fori_loop(0, N_CHUNKS, body, 0)


def kernel(inputs, weight):
    idx = inputs.astype(jnp.int32).reshape(NW, N_CHUNKS, CHUNK)
    out = _gather_kernel(idx, weight)
    return out.reshape(BATCH, SEQ, DIM)


# speedup vs baseline: 1.1133x; 1.1133x over previous
import functools

import jax
import jax.numpy as jnp
from jax import lax
from jax.experimental import pallas as pl
from jax.experimental.pallas import tpu as pltpu
from jax.experimental.pallas import tpu_sc as plsc

BATCH = 4096
SEQ = 200
DIM = 64
B_TOTAL = BATCH * SEQ
NUM_CORES = 2
NUM_SUBCORES = 16
NW = NUM_CORES * NUM_SUBCORES
B_PER_W = B_TOTAL // NW
CHUNK = 128
N_CHUNKS = B_PER_W // CHUNK
SUPER = 512
GPS = SUPER // CHUNK
NSUP = B_PER_W // SUPER

_mesh = plsc.VectorSubcoreMesh(core_axis_name="c", subcore_axis_name="s")


@functools.partial(
    pl.kernel,
    mesh=_mesh,
    out_type=jax.ShapeDtypeStruct((B_TOTAL, DIM), jnp.float32),
    scratch_types=[
        pltpu.VMEM((N_CHUNKS, CHUNK), jnp.int32),
        pltpu.VMEM((SUPER, DIM), jnp.float32),
        pltpu.VMEM((SUPER, DIM), jnp.float32),
        pltpu.SemaphoreType.DMA,
        pltpu.SemaphoreType.DMA,
        pltpu.SemaphoreType.DMA,
    ],
    compiler_params=pltpu.CompilerParams(use_tc_tiling_on_sc=False),
)
def _gather_kernel(idx_hbm, table_hbm, out_hbm, idx_v, rows0, rows1,
                   gsem, ssem0, ssem1):
    wid = lax.axis_index("s") * NUM_CORES + lax.axis_index("c")
    pltpu.sync_copy(idx_hbm.at[wid], idx_v)
    base = wid * B_PER_W

    bufs = (rows0, rows1)
    ssems = (ssem0, ssem1)

    def fire_gathers(g, buf):
        for k in range(GPS):
            pltpu.async_copy(table_hbm.at[idx_v.at[g * GPS + k]],
                             buf.at[pl.ds(k * CHUNK, CHUNK)], gsem)

    def wait_gathers(buf):
        pltpu.make_async_copy(out_hbm.at[pl.ds(0, SUPER)], buf, gsem).wait()

    def wait_store(buf, sem):
        pltpu.make_async_copy(buf, out_hbm.at[pl.ds(0, SUPER)], sem).wait()

    fire_gathers(0, bufs[0])

    def body(i, carry):
        for b in range(2):
            g = 2 * i + b
            nb = 1 - b
            if b == 0:
                @pl.when(i >= 1)
                def _():
                    wait_store(bufs[nb], ssems[nb])
                fire_gathers(g + 1, bufs[nb])
            else:
                @pl.when(i < NSUP // 2 - 1)
                def _():
                    wait_store(bufs[nb], ssems[nb])
                    fire_gathers(g + 1, bufs[nb])
            wait_gathers(bufs[b])
            pltpu.async_copy(bufs[b], out_hbm.at[pl.ds(base + g * SUPER, SUPER)],
                             ssems[b])
        return carry

    lax.fori_loop(0, NSUP // 2, body, 0)

    wait_store(bufs[0], ssems[0])
    wait_store(bufs[1], ssems[1])


def kernel(inputs, weight):
    idx = inputs.astype(jnp.int32).reshape(NW, N_CHUNKS, CHUNK)
    out = _gather_kernel(idx, weight)
    return out.reshape(BATCH, SEQ, DIM)
